# baseline (speedup 1.0000x reference)
import functools

import jax
import jax.numpy as jnp
from jax import lax
from jax.experimental import pallas as pl
from jax.experimental.pallas import tpu as pltpu

NB = 512
W = 128


def _chamfer_kernel(x_ref, y_ref, d1_ref, i1_ref, d2_ref, i2_ref):
    ni = pl.program_id(1)
    x = x_ref[0]
    m = y_ref.shape[2]
    nchunks = m // W

    big = jnp.float32(2**24)
    inf = jnp.float32(jnp.inf)

    xm0 = x[:, 0:1] * -2.0
    xm1 = x[:, 1:2] * -2.0
    xm2 = x[:, 2:3] * -2.0
    x2 = jnp.sum(x * x, axis=1, keepdims=True)

    riota = (lax.broadcasted_iota(jnp.int32, (NB, 1), 0)
             + ni * NB).astype(jnp.float32)
    jlane = lax.broadcasted_iota(jnp.int32, (1, W), 1).astype(jnp.float32)

    @pl.when(ni == 0)
    def _init():
        d2_ref[0] = jnp.full((1, m), inf, jnp.float32)
        i2_ref[0] = jnp.zeros((1, m), jnp.int32)

    def body(c, carry):
        rowbest, rowbesti = carry
        lo = c * W
        yc = y_ref[0, :, pl.ds(lo, W)]
        y2c = jnp.sum(yc * yc, axis=0, keepdims=True)
        d = x2 + (y2c
                  + (xm0 * yc[0:1, :]
                     + (xm1 * yc[1:2, :] + xm2 * yc[2:3, :])))

        mask = d < rowbest
        jvec = jlane + lo.astype(jnp.float32)
        rowbest = jnp.where(mask, d, rowbest)
        rowbesti = jnp.where(mask, jvec, rowbesti)

        m2c = jnp.min(d, axis=0, keepdims=True)
        i2c = jnp.min(jnp.where(d == m2c, riota, big), axis=0,
                      keepdims=True)
        prev_d = d2_ref[0, 0:1, pl.ds(lo, W)]
        prev_i = i2_ref[0, 0:1, pl.ds(lo, W)]
        upd = m2c < prev_d
        d2_ref[0, 0:1, pl.ds(lo, W)] = jnp.where(upd, m2c, prev_d)
        i2_ref[0, 0:1, pl.ds(lo, W)] = jnp.where(
            upd, i2c.astype(jnp.int32), prev_i)
        return rowbest, rowbesti

    rowbest0 = jnp.full((NB, W), inf, jnp.float32)
    rowbesti0 = jnp.zeros((NB, W), jnp.float32)
    rowbest, rowbesti = lax.fori_loop(0, nchunks, body,
                                      (rowbest0, rowbesti0))

    m1 = jnp.min(rowbest, axis=1, keepdims=True)
    i1f = jnp.min(jnp.where(rowbest == m1, rowbesti, big), axis=1,
                  keepdims=True)
    d1_ref[0] = m1
    i1_ref[0] = i1f.astype(jnp.int32)


def kernel(input1, input2):
    b, n, _ = input1.shape
    m = input2.shape[1]
    nblk = n // NB
    y_t = input2.transpose(0, 2, 1)

    d1, i1, d2, i2 = pl.pallas_call(
        _chamfer_kernel,
        grid=(b, nblk),
        in_specs=[
            pl.BlockSpec((1, NB, 3), lambda bi, ni: (bi, ni, 0)),
            pl.BlockSpec((1, 3, m), lambda bi, ni: (bi, 0, 0)),
        ],
        out_specs=[
            pl.BlockSpec((1, NB, 1), lambda bi, ni: (bi * nblk + ni, 0, 0)),
            pl.BlockSpec((1, NB, 1), lambda bi, ni: (bi * nblk + ni, 0, 0)),
            pl.BlockSpec((1, 1, m), lambda bi, ni: (bi, 0, 0)),
            pl.BlockSpec((1, 1, m), lambda bi, ni: (bi, 0, 0)),
        ],
        out_shape=[
            jax.ShapeDtypeStruct((b * nblk, NB, 1), jnp.float32),
            jax.ShapeDtypeStruct((b * nblk, NB, 1), jnp.int32),
            jax.ShapeDtypeStruct((b, 1, m), jnp.float32),
            jax.ShapeDtypeStruct((b, 1, m), jnp.int32),
        ],
        compiler_params=pltpu.CompilerParams(
            dimension_semantics=("parallel", "arbitrary")),
    )(input1, y_t)

    dist1 = d1.reshape(b, n)
    idx1 = i1.reshape(b, n)
    dist2 = d2.reshape(b, m)
    idx2 = i2.reshape(b, m)
    return (dist1, dist2, idx1, idx2)


# speedup vs baseline: 2.2248x; 2.2248x over previous
import jax
import jax.numpy as jnp
from jax import lax
from jax.experimental import pallas as pl
from jax.experimental.pallas import tpu as pltpu

NB = 512
W = 128
SL = 8


def _chamfer_kernel(x_ref, y_ref, d1_ref, i1_ref, d2_ref, i2_ref):
    ni = pl.program_id(1)
    x = x_ref[0]
    m = y_ref.shape[2]
    nchunks = m // W
    nvr = NB // SL

    big = jnp.float32(2**24)
    inf = jnp.float32(jnp.inf)

    xm0 = x[:, 0:1] * -2.0
    xm1 = x[:, 1:2] * -2.0
    xm2 = x[:, 2:3] * -2.0
    x2 = jnp.sum(x * x, axis=1, keepdims=True)

    riota3 = (lax.broadcasted_iota(jnp.int32, (nvr, SL, 1), 0) * SL
              + lax.broadcasted_iota(jnp.int32, (nvr, SL, 1), 1)
              + ni * NB).astype(jnp.float32)
    jlane = lax.broadcasted_iota(jnp.int32, (1, W), 1).astype(jnp.float32)

    rowbest = jnp.full((NB, W), inf, jnp.float32)
    rowbesti = jnp.zeros((NB, W), jnp.float32)
    colparts = []
    colpartis = []

    for c in range(nchunks):
        lo = c * W
        yc = y_ref[0, :, lo:lo + W]
        y2c = jnp.sum(yc * yc, axis=0, keepdims=True)
        d = x2 + (y2c
                  + (xm0 * yc[0:1, :]
                     + (xm1 * yc[1:2, :] + xm2 * yc[2:3, :])))

        mask = d < rowbest
        rowbest = jnp.where(mask, d, rowbest)
        rowbesti = jnp.where(mask, jlane + jnp.float32(lo), rowbesti)

        d3 = d.reshape(nvr, SL, W)
        cp = jnp.min(d3, axis=0)
        cpi = jnp.min(jnp.where(d3 == cp[None], riota3, big), axis=0)
        colparts.append(cp)
        colpartis.append(cpi)

    m1 = jnp.min(rowbest, axis=1, keepdims=True)
    i1f = jnp.min(jnp.where(rowbest == m1, rowbesti, big), axis=1,
                  keepdims=True)
    d1_ref[0] = m1
    i1_ref[0] = i1f.astype(jnp.int32)

    cpf = jnp.concatenate(colparts, axis=1)
    cpfi = jnp.concatenate(colpartis, axis=1)
    m2 = jnp.min(cpf, axis=0, keepdims=True)
    i2f = jnp.min(jnp.where(cpf == m2, cpfi, big), axis=0, keepdims=True)
    i2 = i2f.astype(jnp.int32)

    @pl.when(ni == 0)
    def _init():
        d2_ref[0] = m2
        i2_ref[0] = i2

    @pl.when(ni != 0)
    def _acc():
        prev_d = d2_ref[0]
        prev_i = i2_ref[0]
        upd = m2 < prev_d
        d2_ref[0] = jnp.where(upd, m2, prev_d)
        i2_ref[0] = jnp.where(upd, i2, prev_i)


def kernel(input1, input2):
    b, n, _ = input1.shape
    m = input2.shape[1]
    nblk = n // NB
    y_t = input2.transpose(0, 2, 1)

    d1, i1, d2, i2 = pl.pallas_call(
        _chamfer_kernel,
        grid=(b, nblk),
        in_specs=[
            pl.BlockSpec((1, NB, 3), lambda bi, ni: (bi, ni, 0)),
            pl.BlockSpec((1, 3, m), lambda bi, ni: (bi, 0, 0)),
        ],
        out_specs=[
            pl.BlockSpec((1, NB, 1), lambda bi, ni: (bi * nblk + ni, 0, 0)),
            pl.BlockSpec((1, NB, 1), lambda bi, ni: (bi * nblk + ni, 0, 0)),
            pl.BlockSpec((1, 1, m), lambda bi, ni: (bi, 0, 0)),
            pl.BlockSpec((1, 1, m), lambda bi, ni: (bi, 0, 0)),
        ],
        out_shape=[
            jax.ShapeDtypeStruct((b * nblk, NB, 1), jnp.float32),
            jax.ShapeDtypeStruct((b * nblk, NB, 1), jnp.int32),
            jax.ShapeDtypeStruct((b, 1, m), jnp.float32),
            jax.ShapeDtypeStruct((b, 1, m), jnp.int32),
        ],
        compiler_params=pltpu.CompilerParams(
            dimension_semantics=("parallel", "arbitrary")),
    )(input1, y_t)

    dist1 = d1.reshape(b, n)
    idx1 = i1.reshape(b, n)
    dist2 = d2.reshape(b, m)
    idx2 = i2.reshape(b, m)
    return (dist1, dist2, idx1, idx2)
